# baseline (speedup 1.0000x reference)
import jax
import jax.numpy as jnp
from jax.experimental import pallas as pl
from jax.experimental.pallas import tpu as pltpu

GROUP_SIZE = 128
H_DIM = 2048
I_DIM = 5632
BT = 1024
BI = 512


def _dequant_bf16(q, z, s):
    g, gr, b = q.shape
    w = (q - z[:, None, :]).astype(jnp.bfloat16) * s[:, None, :].astype(jnp.bfloat16)
    return w.reshape(g * gr, b)


def _dequant_kernel(gq_ref, gs_ref, gz_ref, uq_ref, us_ref, uz_ref,
                    dq_ref, ds_ref, dz_ref, wg_ref, wu_ref, wd_ref):
    gh = H_DIM // GROUP_SIZE
    gi = BI // GROUP_SIZE
    wg_ref[...] = _dequant_bf16(gq_ref[...].reshape(gh, GROUP_SIZE, BI),
                                gz_ref[...], gs_ref[...])
    wu_ref[...] = _dequant_bf16(uq_ref[...].reshape(gh, GROUP_SIZE, BI),
                                uz_ref[...], us_ref[...])
    wd_ref[...] = _dequant_bf16(dq_ref[...].reshape(gi, GROUP_SIZE, H_DIM),
                                dz_ref[0], ds_ref[0])


def _swiglu_kernel(x_ref, wg_ref, wu_ref, wd_ref, o_ref):
    i = pl.program_id(1)
    xb = x_ref[...]
    g = jnp.dot(xb, wg_ref[...], preferred_element_type=jnp.float32)
    u = jnp.dot(xb, wu_ref[...], preferred_element_type=jnp.float32)
    h = (g * jax.nn.sigmoid(g) * u).astype(jnp.bfloat16)
    acc = jnp.dot(h, wd_ref[...], preferred_element_type=jnp.float32)

    @pl.when(i == 0)
    def _():
        o_ref[...] = acc

    @pl.when(i > 0)
    def _():
        o_ref[...] += acc


def kernel(x, gate_q, gate_scales, gate_zeros, up_q, up_scales, up_zeros,
           down_q, down_scales, down_zeros):
    T = x.shape[0]
    n_t = T // BT
    n_i = I_DIM // BI
    gh = H_DIM // GROUP_SIZE
    gi = BI // GROUP_SIZE

    ds3 = down_scales.reshape(n_i, gi, H_DIM)
    dz3 = down_zeros.reshape(n_i, gi, H_DIM)

    wg, wu, wd = pl.pallas_call(
        _dequant_kernel,
        grid=(n_i,),
        in_specs=[
            pl.BlockSpec((H_DIM, BI), lambda i: (0, i)),
            pl.BlockSpec((gh, BI), lambda i: (0, i)),
            pl.BlockSpec((gh, BI), lambda i: (0, i)),
            pl.BlockSpec((H_DIM, BI), lambda i: (0, i)),
            pl.BlockSpec((gh, BI), lambda i: (0, i)),
            pl.BlockSpec((gh, BI), lambda i: (0, i)),
            pl.BlockSpec((BI, H_DIM), lambda i: (i, 0)),
            pl.BlockSpec((1, gi, H_DIM), lambda i: (i, 0, 0)),
            pl.BlockSpec((1, gi, H_DIM), lambda i: (i, 0, 0)),
        ],
        out_specs=[
            pl.BlockSpec((H_DIM, BI), lambda i: (0, i)),
            pl.BlockSpec((H_DIM, BI), lambda i: (0, i)),
            pl.BlockSpec((BI, H_DIM), lambda i: (i, 0)),
        ],
        out_shape=[
            jax.ShapeDtypeStruct((H_DIM, I_DIM), jnp.bfloat16),
            jax.ShapeDtypeStruct((H_DIM, I_DIM), jnp.bfloat16),
            jax.ShapeDtypeStruct((I_DIM, H_DIM), jnp.bfloat16),
        ],
        compiler_params=pltpu.CompilerParams(
            dimension_semantics=("parallel",),
            vmem_limit_bytes=56 * 1024 * 1024,
        ),
        name="gptq_dequant_bf16",
    )(gate_q, gate_scales, gate_zeros, up_q, up_scales, up_zeros,
      down_q, ds3, dz3)

    xb = x.astype(jnp.bfloat16)

    out = pl.pallas_call(
        _swiglu_kernel,
        grid=(n_t, n_i),
        in_specs=[
            pl.BlockSpec((BT, H_DIM), lambda t, i: (t, 0)),
            pl.BlockSpec((H_DIM, BI), lambda t, i: (0, i)),
            pl.BlockSpec((H_DIM, BI), lambda t, i: (0, i)),
            pl.BlockSpec((BI, H_DIM), lambda t, i: (i, 0)),
        ],
        out_specs=pl.BlockSpec((BT, H_DIM), lambda t, i: (t, 0)),
        out_shape=jax.ShapeDtypeStruct((T, H_DIM), jnp.float32),
        compiler_params=pltpu.CompilerParams(
            dimension_semantics=("parallel", "arbitrary"),
            vmem_limit_bytes=56 * 1024 * 1024,
        ),
        name="swiglu_bf16_mlp",
    )(xb, wg, wu, wd)
    return out


# speedup vs baseline: 1.2629x; 1.2629x over previous
import jax
import jax.numpy as jnp
from jax.experimental import pallas as pl
from jax.experimental.pallas import tpu as pltpu

GROUP_SIZE = 128
H_DIM = 2048
I_DIM = 5632
BT = 1024
BI = 512


def _dequant_bf16(q, z, s):
    g, gr, b = q.shape
    w = (q - z[:, None, :]).astype(jnp.bfloat16) * s[:, None, :].astype(jnp.bfloat16)
    return w.reshape(g * gr, b)


def _mlp_kernel(x_ref, gq_ref, gs_ref, gz_ref, uq_ref, us_ref, uz_ref,
                dq_ref, ds_ref, dz_ref, o_ref):
    i = pl.program_id(1)
    not_first = i > 0

    gh = H_DIM // GROUP_SIZE
    gi = BI // GROUP_SIZE
    xb = x_ref[...]

    wg = _dequant_bf16(gq_ref[...].reshape(gh, GROUP_SIZE, BI),
                       gz_ref[...], gs_ref[...])
    g = jnp.dot(xb, wg, preferred_element_type=jnp.float32)

    wu = _dequant_bf16(uq_ref[...].reshape(gh, GROUP_SIZE, BI),
                       uz_ref[...], us_ref[...])
    u = jnp.dot(xb, wu, preferred_element_type=jnp.float32)

    wd = _dequant_bf16(dq_ref[...].reshape(gi, GROUP_SIZE, H_DIM),
                       dz_ref[0], ds_ref[0])

    h = (g * jax.nn.sigmoid(g) * u).astype(jnp.bfloat16)

    for n in range(H_DIM // 512):
        sl = slice(n * 512, (n + 1) * 512)
        acc = jnp.dot(h, wd[:, sl], preferred_element_type=jnp.float32)
        prev = jnp.where(not_first, o_ref[:, sl], 0.0)
        o_ref[:, sl] = prev + acc


def kernel(x, gate_q, gate_scales, gate_zeros, up_q, up_scales, up_zeros,
           down_q, down_scales, down_zeros):
    T = x.shape[0]
    n_t = T // BT
    n_i = I_DIM // BI
    gh = H_DIM // GROUP_SIZE
    gi = BI // GROUP_SIZE

    ds3 = down_scales.reshape(n_i, gi, H_DIM)
    dz3 = down_zeros.reshape(n_i, gi, H_DIM)
    xb = x.astype(jnp.bfloat16)

    out = pl.pallas_call(
        _mlp_kernel,
        grid=(n_t, n_i),
        in_specs=[
            pl.BlockSpec((BT, H_DIM), lambda t, i: (t, 0)),
            pl.BlockSpec((H_DIM, BI), lambda t, i: (0, i)),
            pl.BlockSpec((gh, BI), lambda t, i: (0, i)),
            pl.BlockSpec((gh, BI), lambda t, i: (0, i)),
            pl.BlockSpec((H_DIM, BI), lambda t, i: (0, i)),
            pl.BlockSpec((gh, BI), lambda t, i: (0, i)),
            pl.BlockSpec((gh, BI), lambda t, i: (0, i)),
            pl.BlockSpec((BI, H_DIM), lambda t, i: (i, 0)),
            pl.BlockSpec((1, gi, H_DIM), lambda t, i: (i, 0, 0)),
            pl.BlockSpec((1, gi, H_DIM), lambda t, i: (i, 0, 0)),
        ],
        out_specs=pl.BlockSpec((BT, H_DIM), lambda t, i: (t, 0)),
        out_shape=jax.ShapeDtypeStruct((T, H_DIM), jnp.float32),
        compiler_params=pltpu.CompilerParams(
            dimension_semantics=("parallel", "arbitrary"),
            vmem_limit_bytes=56 * 1024 * 1024,
        ),
        name="moe_shared_expert_mlp",
    )(xb, gate_q, gate_scales, gate_zeros, up_q, up_scales, up_zeros,
      down_q, ds3, dz3)
    return out
